# baseline (speedup 1.0000x reference)
import functools

import jax
import jax.numpy as jnp
from jax import lax
from jax.experimental import pallas as pl
from jax.experimental.pallas import tpu as pltpu
from jax.experimental.pallas import tpu_sc as plsc

D = 1024
L = 16
EPS = 1e-05
SEQ = 4096
C = 16
NB = D // L

_GATHER_DNUMS = lax.GatherDimensionNumbers(
    offset_dims=(), collapsed_slice_dims=(0,), start_index_map=(0,))


def _lane_pick(v, idx):
    return lax.gather(v, idx[:, None], _GATHER_DNUMS, slice_sizes=(1,),
                      mode=lax.GatherScatterMode.PROMISE_IN_BOUNDS)


def _rsqrt(v):
    i = plsc.bitcast(v, jnp.int32)
    y = plsc.bitcast(jnp.int32(0x5F3759DF) - (i >> 1), jnp.float32)
    for _ in range(3):
        y = y * (1.5 - 0.5 * v * y * y)
    return y


@functools.lru_cache(maxsize=None)
def _make_sc_kernel(n_rows):
    info = plsc.get_sparse_core_info()
    nw = info.num_cores * info.num_subcores
    per_w = n_rows // nw
    n_g = per_w // C
    mesh = plsc.VectorSubcoreMesh(core_axis_name="c", subcore_axis_name="s")

    @functools.partial(
        pl.kernel,
        mesh=mesh,
        out_type=jax.ShapeDtypeStruct((n_rows, D), jnp.float32),
        compiler_params=pltpu.CompilerParams(needs_layout_passes=False),
        scratch_types=[
            pltpu.VMEM((per_w,), jnp.int32),
            pltpu.VMEM((C, D), jnp.float32),
            pltpu.VMEM((C, D), jnp.float32),
            pltpu.VMEM((C, D), jnp.float32),
            pltpu.VMEM((C, D), jnp.float32),
            pltpu.VMEM((C, D), jnp.float32),
            pltpu.VMEM((C, D), jnp.float32),
            pltpu.VMEM((D,), jnp.float32),
            pltpu.VMEM((D,), jnp.float32),
            pltpu.VMEM((D,), jnp.float32),
            pltpu.SemaphoreType.DMA,
            pltpu.SemaphoreType.DMA,
            pltpu.SemaphoreType.DMA,
            pltpu.SemaphoreType.DMA,
            pltpu.SemaphoreType.DMA,
            pltpu.SemaphoreType.DMA,
        ],
    )
    def k(ids_hbm, word_hbm, pos_hbm, type_hbm, w_hbm, b_hbm, out_hbm,
          idx_v, word_b0, word_b1, pos_b0, pos_b1, out_b0, out_b1,
          type_v, w_v, b_v, semw0, semw1, semp0, semp1, semo0, semo1):
        word_bufs = (word_b0, word_b1)
        pos_bufs = (pos_b0, pos_b1)
        out_bufs = (out_b0, out_b1)
        semw = (semw0, semw1)
        semp = (semp0, semp1)
        semo = (semo0, semo1)

        wid = lax.axis_index("s") * info.num_cores + lax.axis_index("c")
        base = wid * per_w
        s0 = lax.rem(base, SEQ)
        pltpu.sync_copy(ids_hbm.at[pl.ds(base, per_w)], idx_v)
        pltpu.sync_copy(type_hbm, type_v)
        pltpu.sync_copy(w_hbm, w_v)
        pltpu.sync_copy(b_hbm, b_v)

        lanes = lax.iota(jnp.int32, L)
        cjj = [jnp.full((L,), jj, jnp.int32) for jj in range(L)]
        zf = jnp.zeros((L,), jnp.float32)
        zi = jnp.zeros((L,), jnp.int32)

        def word_gather(g, p):
            row0 = pl.multiple_of(g * C, C)
            return pltpu.make_async_copy(
                word_hbm.at[idx_v.at[pl.ds(row0, C)]], word_bufs[p], semw[p])

        def pos_copy(g, p):
            row0 = pl.multiple_of(g * C, C)
            return pltpu.make_async_copy(
                pos_hbm.at[pl.ds(s0 + row0, C)], pos_bufs[p], semp[p])

        def out_copy(g, p):
            row0 = pl.multiple_of(base + g * C, C)
            return pltpu.make_async_copy(
                out_bufs[p], out_hbm.at[pl.ds(row0, C)], semo[p])

        def start_gather(g, p):
            word_gather(g, p).start()
            pos_copy(g, p).start()

        start_gather(0, 0)
        start_gather(1, 1)

        def process(g, p):
            @pl.when(g >= 2)
            def _():
                out_copy(g, p).wait()

            word_gather(g, p).wait()
            pos_copy(g, p).wait()

            def p1_block(bi, carry):
                vs, vq, colv = carry
                tch = type_v[pl.ds(pl.multiple_of(bi * L, L), L)]
                for jj in range(L):
                    xw = plsc.load_gather(word_bufs[p], [lanes, colv])
                    xp = plsc.load_gather(pos_bufs[p], [lanes, colv])
                    x = xw + xp + _lane_pick(tch, cjj[jj])
                    plsc.store_scatter(out_bufs[p], [lanes, colv], x)
                    vs = vs + x
                    vq = vq + x * x
                    colv = colv + 1
                return vs, vq, colv

            vs, vq, _ = lax.fori_loop(0, NB, p1_block, (zf, zf, zi))

            @pl.when(g + 2 < n_g)
            def _():
                start_gather(g + 2, p)

            mean = vs * (1.0 / D)
            var = vq * (1.0 / D) - mean * mean
            rstd = _rsqrt(var + EPS)

            def p2_block(bi, colv):
                wch = w_v[pl.ds(pl.multiple_of(bi * L, L), L)]
                bch = b_v[pl.ds(pl.multiple_of(bi * L, L), L)]
                for jj in range(L):
                    x = plsc.load_gather(out_bufs[p], [lanes, colv])
                    y = ((x - mean) * rstd * _lane_pick(wch, cjj[jj])
                         + _lane_pick(bch, cjj[jj]))
                    plsc.store_scatter(out_bufs[p], [lanes, colv], y)
                    colv = colv + 1
                return colv

            lax.fori_loop(0, NB, p2_block, zi)
            out_copy(g, p).start()

        def outer(go, carry):
            process(2 * go, 0)
            process(2 * go + 1, 1)
            return carry

        lax.fori_loop(0, n_g // 2, outer, 0)
        out_copy(n_g - 2, 0).wait()
        out_copy(n_g - 1, 1).wait()

    return k


def kernel(input_ids, word_table, pos_table, type_table, ln_weight, ln_bias):
    b, s = input_ids.shape
    ids_flat = jnp.reshape(input_ids.astype(jnp.int32), (b * s,))
    type_row = jnp.reshape(type_table, (D,))
    k = _make_sc_kernel(b * s)
    out = k(ids_flat, word_table, pos_table, type_row, ln_weight, ln_bias)
    return jnp.reshape(out, (b, s, D))
